# baseline (speedup 1.0000x reference)
import functools

import jax
import jax.numpy as jnp
from jax import lax
from jax.experimental import pallas as pl
from jax.experimental.pallas import tpu as pltpu
from jax.experimental.pallas import tpu_sc as plsc

N = 320
NUM_NODES = N * N
NW = 32
ROWS_W = N // NW
NODES_W = ROWS_W * N
VALS_W = NODES_W * 4
SLAB_ROWS = 24
VPR = (N * 4) // 16

_MESH = plsc.VectorSubcoreMesh(core_axis_name="c", subcore_axis_name="s",
                               num_cores=2, num_subcores=16)


@functools.partial(
    pl.kernel,
    out_type=(
        jax.ShapeDtypeStruct((NUM_NODES * 4,), jnp.float32),
        jax.ShapeDtypeStruct((NUM_NODES * 4,), jnp.int32),
        jax.ShapeDtypeStruct((NUM_NODES * 4,), jnp.int32),
    ),
    mesh=_MESH,
    compiler_params=pltpu.CompilerParams(needs_layout_passes=False),
    scratch_types=(
        pltpu.VMEM((SLAB_ROWS, N), jnp.float32),
        pltpu.VMEM((VALS_W,), jnp.float32),
        pltpu.VMEM((VALS_W,), jnp.int32),
        pltpu.VMEM((VALS_W,), jnp.int32),
    ),
)
def _grid_adjacency_sc(grid_hbm, vals_hbm, rows_hbm, cols_hbm,
                       slab_v, vals_v, rows_v, cols_v):
    wid = lax.axis_index("s") * 2 + lax.axis_index("c")
    i0 = wid * ROWS_W
    s = pl.multiple_of(
        jnp.clip(((i0 - 1) // 8) * 8, 0, N - SLAB_ROWS), 8)

    pltpu.sync_copy(grid_hbm.at[pl.ds(s, SLAB_ROWS)], slab_v)

    lane = lax.iota(jnp.int32, 16)
    k = lane & 3
    l4 = lane >> 2
    di = jnp.where(k == 0, 1, jnp.where(k == 1, -1, 0))
    dj = jnp.where(k == 2, 1, jnp.where(k == 3, -1, 0))

    def row_body(r, _):
        i = i0 + r
        ti = i + di
        ok_i = (ti >= 0) & (ti < N)
        ti_loc = jnp.clip(ti, 0, N - 1) - s
        coloff = ti * N
        vbase = i * N

        def vec_body(jj, _):
            jv = jj * 4 + l4
            tj = jv + dj
            m = ok_i & (tj >= 0) & (tj < N)
            val = plsc.load_gather(slab_v, [ti_loc, jnp.clip(tj, 0, N - 1)])
            lbase = r * (N * 4) + jj * 16
            sl = pl.ds(lbase, 16)
            vals_v[sl] = jnp.where(m, val, 0.0)
            rows_v[sl] = jnp.where(m, vbase + jv, 0)
            cols_v[sl] = jnp.where(m, coloff + tj, 0)
            return 0

        lax.fori_loop(0, VPR, vec_body, 0)
        return 0

    lax.fori_loop(0, ROWS_W, row_body, 0)

    osl = pl.ds(wid * VALS_W, VALS_W)
    pltpu.sync_copy(vals_v, vals_hbm.at[osl])
    pltpu.sync_copy(rows_v, rows_hbm.at[osl])
    pltpu.sync_copy(cols_v, cols_hbm.at[osl])


def kernel(grid):
    vals, rows, cols = _grid_adjacency_sc(grid)
    return vals, jnp.stack([rows, cols], axis=1)


# speedup vs baseline: 1.0400x; 1.0400x over previous
import functools

import jax
import jax.numpy as jnp
from jax import lax
from jax.experimental import pallas as pl
from jax.experimental.pallas import tpu as pltpu
from jax.experimental.pallas import tpu_sc as plsc

N = 320
NUM_NODES = N * N
NW = 32
ROWS_W = N // NW
NODES_W = ROWS_W * N
VALS_W = NODES_W * 4
SLAB_ROWS = 24
VPR = (N * 4) // 16

_MESH = plsc.VectorSubcoreMesh(core_axis_name="c", subcore_axis_name="s",
                               num_cores=2, num_subcores=16)


@functools.partial(
    pl.kernel,
    out_type=(
        jax.ShapeDtypeStruct((NUM_NODES * 4,), jnp.float32),
        jax.ShapeDtypeStruct((NUM_NODES * 4,), jnp.int32),
        jax.ShapeDtypeStruct((NUM_NODES * 4,), jnp.int32),
    ),
    mesh=_MESH,
    compiler_params=pltpu.CompilerParams(needs_layout_passes=False),
    scratch_types=(
        pltpu.VMEM((SLAB_ROWS, N), jnp.float32),
        pltpu.VMEM((VALS_W,), jnp.float32),
        pltpu.VMEM((VALS_W,), jnp.int32),
        pltpu.VMEM((VALS_W,), jnp.int32),
        pltpu.SemaphoreType.DMA,
    ),
)
def _grid_adjacency_sc(grid_hbm, vals_hbm, rows_hbm, cols_hbm,
                       slab_v, vals_v, rows_v, cols_v, sem):
    wid = lax.axis_index("s") * 2 + lax.axis_index("c")
    i0 = wid * ROWS_W
    s = pl.multiple_of(
        jnp.clip(((i0 - 1) // 8) * 8, 0, N - SLAB_ROWS), 8)

    pltpu.sync_copy(grid_hbm.at[pl.ds(s, SLAB_ROWS)], slab_v)

    lane = lax.iota(jnp.int32, 16)
    k = lane & 3
    l4 = lane >> 2
    di = jnp.where(k == 0, 1, jnp.where(k == 1, -1, 0))
    dj = jnp.where(k == 2, 1, jnp.where(k == 3, -1, 0))

    copies = []
    for r in range(ROWS_W):
        i = i0 + r
        ti = i + di
        ok_i = (ti >= 0) & (ti < N)
        ti_loc = jnp.clip(ti, 0, N - 1) - s
        coloff = ti * N
        vbase = i * N

        @plsc.parallel_loop(0, VPR, unroll=4)
        def vec_body(jj):
            jv = jj * 4 + l4
            tj = jv + dj
            m = ok_i & (tj >= 0) & (tj < N)
            val = plsc.load_gather(slab_v, [ti_loc, jnp.clip(tj, 0, N - 1)])
            sl = pl.ds(r * (N * 4) + jj * 16, 16)
            vals_v[sl] = jnp.where(m, val, 0.0)
            rows_v[sl] = jnp.where(m, vbase + jv, 0)
            cols_v[sl] = jnp.where(m, coloff + tj, 0)

        lsl = pl.ds(r * (N * 4), N * 4)
        osl = pl.ds(wid * VALS_W + r * (N * 4), N * 4)
        copies.append(pltpu.async_copy(vals_v.at[lsl], vals_hbm.at[osl], sem))
        copies.append(pltpu.async_copy(rows_v.at[lsl], rows_hbm.at[osl], sem))
        copies.append(pltpu.async_copy(cols_v.at[lsl], cols_hbm.at[osl], sem))

    for c in copies:
        c.wait()


def kernel(grid):
    vals, rows, cols = _grid_adjacency_sc(grid)
    return vals, jnp.stack([rows, cols], axis=1)
